# Initial kernel scaffold; baseline (speedup 1.0000x reference)
import jax
import jax.numpy as jnp
from jax.experimental import pallas as pl


def kernel(x, edge_index, edge_weights, W1, b1, g1, bt1, W2, b2, g2, bt2):
    raise NotImplementedError("write your pallas kernel here")


# speedup vs baseline: 3.2189x; 3.2189x over previous
import dataclasses
import functools

import jax
import jax.numpy as jnp
from jax import lax
from jax.experimental import pallas as pl
from jax.experimental.pallas import tpu as pltpu
from jax.experimental.pallas import tpu_sc as plsc

_NC = 2
_NS = 16
_CH = 128


def _linear_relu(x, W, b):
    n, d = x.shape
    blk = 1000

    def body(x_ref, w_ref, b_ref, o_ref):
        h = jnp.dot(x_ref[...], w_ref[...],
                    preferred_element_type=jnp.float32,
                    precision=jax.lax.Precision.HIGHEST)
        o_ref[...] = jnp.maximum(h + b_ref[...], 0.0)

    return pl.pallas_call(
        body,
        grid=(n // blk,),
        in_specs=[
            pl.BlockSpec((blk, d), lambda i: (i, 0)),
            pl.BlockSpec((d, d), lambda i: (0, 0)),
            pl.BlockSpec((1, d), lambda i: (0, 0)),
        ],
        out_specs=pl.BlockSpec((blk, d), lambda i: (i, 0)),
        out_shape=jax.ShapeDtypeStruct((n, d), jnp.float32),
    )(x, W, b.reshape(1, d))


def _combine_ln(h, a0, a1, g, bt):
    n, d = h.shape
    blk = 1000

    def body(h_ref, a0_ref, a1_ref, g_ref, bt_ref, o_ref):
        s = h_ref[...] + a0_ref[...] + a1_ref[...]
        mu = jnp.mean(s, axis=-1, keepdims=True)
        var = jnp.mean((s - mu) ** 2, axis=-1, keepdims=True)
        o_ref[...] = (s - mu) * jax.lax.rsqrt(var + 1e-5) * g_ref[...] + bt_ref[...]

    return pl.pallas_call(
        body,
        grid=(n // blk,),
        in_specs=[
            pl.BlockSpec((blk, d), lambda i: (i, 0)),
            pl.BlockSpec((blk, d), lambda i: (i, 0)),
            pl.BlockSpec((blk, d), lambda i: (i, 0)),
            pl.BlockSpec((1, d), lambda i: (0, 0)),
            pl.BlockSpec((1, d), lambda i: (0, 0)),
        ],
        out_specs=pl.BlockSpec((blk, d), lambda i: (i, 0)),
        out_shape=jax.ShapeDtypeStruct((n, d), jnp.float32),
    )(h, a0, a1, g.reshape(1, d), bt.reshape(1, d))


def _edge_pass(h, src, dst, w, zeros):
    n, d = h.shape
    e_pad = src.shape[0]
    nw = _NC * _NS
    per_w = e_pad // nw
    n_chunks = per_w // _CH
    slab = ((n + _NS - 1) // _NS + 7) // 8 * 8
    last_slab = n - (_NS - 1) * slab
    assert last_slab > 0 and last_slab % 8 == 0

    mesh = plsc.VectorSubcoreMesh(core_axis_name="c", subcore_axis_name="s")

    cp = pltpu.CompilerParams()
    if "needs_layout_passes" in pltpu.CompilerParams.__dataclass_fields__:
        cp = dataclasses.replace(cp, needs_layout_passes=False)

    @functools.partial(
        pl.kernel,
        compiler_params=cp,
        out_type=jax.ShapeDtypeStruct((_NC, n, d), jnp.float32),
        mesh=mesh,
        scratch_types=[
            pltpu.VMEM((_CH,), jnp.int32),
            pltpu.VMEM((_CH,), jnp.int32),
            pltpu.VMEM((_CH,), jnp.float32),
            pltpu.VMEM((_CH, d), jnp.float32),
            pltpu.VMEM_SHARED((n, d), jnp.float32),
        ],
    )
    def ek(h_hbm, src_hbm, dst_hbm, w_hbm, z_hbm, out_hbm,
           src_v, dst_v, w_v, rows_v, acc_sh):
        cid = lax.axis_index("c")
        sid = lax.axis_index("s")
        wid = cid * _NS + sid

        base = sid * slab

        @pl.when(sid < _NS - 1)
        def _():
            pltpu.sync_copy(z_hbm.at[pl.ds(base, slab)],
                            acc_sh.at[pl.ds(base, slab)])

        @pl.when(sid == _NS - 1)
        def _():
            pltpu.sync_copy(z_hbm.at[pl.ds((_NS - 1) * slab, last_slab)],
                            acc_sh.at[pl.ds((_NS - 1) * slab, last_slab)])

        plsc.subcore_barrier()

        ebase0 = wid * per_w

        @pl.loop(0, n_chunks)
        def _(k):
            eb = ebase0 + k * _CH
            pltpu.sync_copy(src_hbm.at[pl.ds(eb, _CH)], src_v)
            pltpu.sync_copy(dst_hbm.at[pl.ds(eb, _CH)], dst_v)
            pltpu.sync_copy(w_hbm.at[pl.ds(eb, _CH)], w_v)
            pltpu.sync_copy(h_hbm.at[src_v], rows_v)

            @pl.loop(0, _CH)
            def _(i):
                idx = jnp.full((16,), i, jnp.int32)
                wgt = plsc.load_gather(w_v, [idx])
                for j in range(d // 16):
                    sl = (i, pl.ds(16 * j, 16))
                    rows_v[sl] = rows_v[sl] * wgt

            pltpu.sync_copy(rows_v, acc_sh.at[dst_v], add=True)

        plsc.subcore_barrier()

        @pl.when(sid < _NS - 1)
        def _():
            pltpu.sync_copy(acc_sh.at[pl.ds(base, slab)],
                            out_hbm.at[cid, pl.ds(base, slab)])

        @pl.when(sid == _NS - 1)
        def _():
            pltpu.sync_copy(acc_sh.at[pl.ds((_NS - 1) * slab, last_slab)],
                            out_hbm.at[cid, pl.ds((_NS - 1) * slab, last_slab)])

    return ek(h, src, dst, w, zeros)


def _gcn_layer(x, src, dst, w, zeros, W, b, g, bt):
    h = _linear_relu(x, W, b)
    acc = _edge_pass(h, src, dst, w, zeros)
    return _combine_ln(h, acc[0], acc[1], g, bt)


def kernel(x, edge_index, edge_weights, W1, b1, g1, bt1, W2, b2, g2, bt2):
    n, d = x.shape
    src = edge_index[0].astype(jnp.int32)
    dst = edge_index[1].astype(jnp.int32)
    w = edge_weights.astype(jnp.float32)

    e = src.shape[0]
    unit = _NC * _NS * _CH
    e_pad = ((e + unit - 1) // unit) * unit
    pad = e_pad - e
    if pad:
        src = jnp.concatenate([src, jnp.zeros((pad,), jnp.int32)])
        dst = jnp.concatenate([dst, jnp.zeros((pad,), jnp.int32)])
        w = jnp.concatenate([w, jnp.zeros((pad,), jnp.float32)])

    zeros = jnp.zeros((n, d), jnp.float32)

    h = _gcn_layer(x, src, dst, w, zeros, W1, b1, g1, bt1)
    h = _gcn_layer(h, src, dst, w, zeros, W2, b2, g2, bt2)
    return h
